# baseline (speedup 1.0000x reference)
import functools

import jax
import jax.numpy as jnp
from jax import lax
from jax.experimental import pallas as pl
from jax.experimental.pallas import tpu as pltpu
from jax.experimental.pallas import tpu_sc as plsc

VOCAB = 1_000_000
D = 64
BATCH = 4096
LENGTH = 200
B_TOTAL = BATCH * LENGTH

NC = 2
NS = 16
NW = NC * NS
B_PER_W = B_TOTAL // NW
CHUNK = 800
NCHUNK = B_PER_W // CHUNK


def _gather_body(idx_hbm, table_hbm, out_hbm, idx_v, rows_v, sem):
    wid = lax.axis_index("s") * NC + lax.axis_index("c")
    base = wid * B_PER_W

    def step(c, _):
        off = base + c * CHUNK
        pltpu.sync_copy(idx_hbm.at[pl.ds(off, CHUNK)], idx_v)
        pltpu.async_copy(table_hbm.at[idx_v], rows_v, sem).wait()
        pltpu.sync_copy(rows_v, out_hbm.at[pl.ds(off, CHUNK)])
        return ()

    lax.fori_loop(0, NCHUNK, step, ())


_gather = functools.partial(
    pl.kernel,
    out_type=jax.ShapeDtypeStruct((B_TOTAL, D), jnp.float32),
    mesh=plsc.VectorSubcoreMesh(core_axis_name="c", subcore_axis_name="s"),
    compiler_params=pltpu.CompilerParams(use_tc_tiling_on_sc=False),
    scratch_types=[
        pltpu.VMEM((CHUNK,), jnp.int32),
        pltpu.VMEM((CHUNK, D), jnp.float32),
        pltpu.SemaphoreType.DMA,
    ],
)(_gather_body)


@jax.jit
def kernel(inputs, shared_weights):
    idx = inputs.reshape(B_TOTAL).astype(jnp.int32)
    out = _gather(idx, shared_weights)
    return out.reshape(BATCH, LENGTH, D)


# speedup vs baseline: 1.0217x; 1.0217x over previous
import functools

import jax
import jax.numpy as jnp
from jax import lax
from jax.experimental import pallas as pl
from jax.experimental.pallas import tpu as pltpu
from jax.experimental.pallas import tpu_sc as plsc

VOCAB = 1_000_000
D = 64
BATCH = 4096
LENGTH = 200
B_TOTAL = BATCH * LENGTH

NC = 2
NS = 16
NW = NC * NS
B_PER_W = B_TOTAL // NW
CHUNK = 800
NCHUNK = B_PER_W // CHUNK
NPAIR = NCHUNK // 2


def _gather_body(idx_hbm, table_hbm, out_hbm, idx_v, rows_v, gsem, wsem):
    wid = lax.axis_index("s") * NC + lax.axis_index("c")
    base = wid * B_PER_W

    pltpu.sync_copy(idx_hbm.at[pl.ds(base, B_PER_W)], idx_v)

    def gather(c, buf):
        return pltpu.make_async_copy(
            table_hbm.at[idx_v.at[pl.ds(c * CHUNK, CHUNK)]],
            rows_v.at[buf],
            gsem,
        )

    def write(c, buf):
        return pltpu.make_async_copy(
            rows_v.at[buf],
            out_hbm.at[pl.ds(base + c * CHUNK, CHUNK)],
            wsem,
        )

    for j in range(2):
        g = gather(j, j)
        g.start()
        g.wait()
        write(j, j).start()

    def pair(p, _):
        for j in range(2):
            c = 2 * p + j
            g = gather(c, j)
            g.start()
            write(c - 2, j).wait()
            g.wait()
            write(c, j).start()
        return ()

    lax.fori_loop(1, NPAIR, pair, ())

    write(NCHUNK - 2, 0).wait()
    write(NCHUNK - 1, 1).wait()


_gather = functools.partial(
    pl.kernel,
    out_type=jax.ShapeDtypeStruct((B_TOTAL, D), jnp.float32),
    mesh=plsc.VectorSubcoreMesh(core_axis_name="c", subcore_axis_name="s"),
    compiler_params=pltpu.CompilerParams(use_tc_tiling_on_sc=False),
    scratch_types=[
        pltpu.VMEM((B_PER_W,), jnp.int32),
        pltpu.VMEM((2, CHUNK, D), jnp.float32),
        pltpu.SemaphoreType.DMA,
        pltpu.SemaphoreType.DMA,
    ],
)(_gather_body)


@jax.jit
def kernel(inputs, shared_weights):
    idx = inputs.reshape(B_TOTAL).astype(jnp.int32)
    out = _gather(idx, shared_weights)
    return out.reshape(BATCH, LENGTH, D)


# speedup vs baseline: 1.0217x; 1.0000x over previous
import functools

import jax
import jax.numpy as jnp
from jax import lax
from jax.experimental import pallas as pl
from jax.experimental.pallas import tpu as pltpu
from jax.experimental.pallas import tpu_sc as plsc

VOCAB = 1_000_000
D = 64
BATCH = 4096
LENGTH = 200
B_TOTAL = BATCH * LENGTH

NC = 2
NS = 16
NW = NC * NS
B_PER_W = B_TOTAL // NW
CHUNK = 800
NCHUNK = B_PER_W // CHUNK
NPAIR = NCHUNK // 2
KSPLIT = 4


def _gather_body(idx_hbm, table_hbm, out_hbm, idx_v, rows_v, gsem, wsem):
    wid = lax.axis_index("s") * NC + lax.axis_index("c")
    base = wid * B_PER_W

    pltpu.sync_copy(idx_hbm.at[pl.ds(base, B_PER_W)], idx_v)

    def gather_part(c, buf, k):
        sub = CHUNK // KSPLIT
        return pltpu.make_async_copy(
            table_hbm.at[idx_v.at[pl.ds(c * CHUNK + k * sub, sub)]],
            rows_v.at[buf, pl.ds(k * sub, sub)],
            gsem,
        )

    def write(c, buf):
        return pltpu.make_async_copy(
            rows_v.at[buf],
            out_hbm.at[pl.ds(base + c * CHUNK, CHUNK)],
            wsem,
        )

    for j in range(2):
        for k in range(KSPLIT):
            gather_part(j, j, k).start()
        for k in range(KSPLIT):
            gather_part(j, j, k).wait()
        write(j, j).start()

    def pair(p, _):
        for j in range(2):
            c = 2 * p + j
            write(c - 2, j).wait()
            for k in range(KSPLIT):
                gather_part(c, j, k).start()
            for k in range(KSPLIT):
                gather_part(c, j, k).wait()
            write(c, j).start()
        return ()

    lax.fori_loop(1, NPAIR, pair, ())

    write(NCHUNK - 2, 0).wait()
    write(NCHUNK - 1, 1).wait()


_gather = functools.partial(
    pl.kernel,
    out_type=jax.ShapeDtypeStruct((B_TOTAL, D), jnp.float32),
    mesh=plsc.VectorSubcoreMesh(core_axis_name="c", subcore_axis_name="s"),
    compiler_params=pltpu.CompilerParams(use_tc_tiling_on_sc=False),
    scratch_types=[
        pltpu.VMEM((B_PER_W,), jnp.int32),
        pltpu.VMEM((2, CHUNK, D), jnp.float32),
        pltpu.SemaphoreType.DMA,
        pltpu.SemaphoreType.DMA,
    ],
)(_gather_body)


@jax.jit
def kernel(inputs, shared_weights):
    idx = inputs.reshape(B_TOTAL).astype(jnp.int32)
    out = _gather(idx, shared_weights)
    return out.reshape(BATCH, LENGTH, D)


# speedup vs baseline: 1.0228x; 1.0011x over previous
import functools

import jax
import jax.numpy as jnp
from jax import lax
from jax.experimental import pallas as pl
from jax.experimental.pallas import tpu as pltpu
from jax.experimental.pallas import tpu_sc as plsc

VOCAB = 1_000_000
D = 64
BATCH = 4096
LENGTH = 200
B_TOTAL = BATCH * LENGTH

NC = 2
NS = 16
NW = NC * NS
B_PER_W = B_TOTAL // NW
CHUNK = 800
NCHUNK = B_PER_W // CHUNK
NPAIR = NCHUNK // 2
KSPLIT = 4


ROWS_PER_CHUNK = CHUNK // LENGTH
ROWS_PER_W = B_PER_W // LENGTH


def _gather_body(idx_hbm, table_hbm, out_hbm3d, idx_v, rows_v, gsem, wsem):
    wid = lax.axis_index("s") * NC + lax.axis_index("c")
    base = wid * B_PER_W

    pltpu.sync_copy(idx_hbm.at[pl.ds(base, B_PER_W)], idx_v)

    def gather_part(c, buf, k):
        return pltpu.make_async_copy(
            table_hbm.at[idx_v.at[pl.ds(c * CHUNK + k * LENGTH, LENGTH)]],
            rows_v.at[buf, k],
            gsem,
        )

    def write(c, buf):
        row0 = wid * ROWS_PER_W + c * ROWS_PER_CHUNK
        return pltpu.make_async_copy(
            rows_v.at[buf],
            out_hbm3d.at[pl.ds(row0, ROWS_PER_CHUNK)],
            wsem,
        )

    for j in range(2):
        for k in range(KSPLIT):
            gather_part(j, j, k).start()
        for k in range(KSPLIT):
            gather_part(j, j, k).wait()
        write(j, j).start()

    def pair(p, _):
        for j in range(2):
            c = 2 * p + j
            write(c - 2, j).wait()
            for k in range(KSPLIT):
                gather_part(c, j, k).start()
            for k in range(KSPLIT):
                gather_part(c, j, k).wait()
            write(c, j).start()
        return ()

    lax.fori_loop(1, NPAIR, pair, ())

    write(NCHUNK - 2, 0).wait()
    write(NCHUNK - 1, 1).wait()


_gather = functools.partial(
    pl.kernel,
    out_type=jax.ShapeDtypeStruct((BATCH, LENGTH, D), jnp.float32),
    mesh=plsc.VectorSubcoreMesh(core_axis_name="c", subcore_axis_name="s"),
    compiler_params=pltpu.CompilerParams(use_tc_tiling_on_sc=False),
    scratch_types=[
        pltpu.VMEM((B_PER_W,), jnp.int32),
        pltpu.VMEM((2, ROWS_PER_CHUNK, LENGTH, D), jnp.float32),
        pltpu.SemaphoreType.DMA,
        pltpu.SemaphoreType.DMA,
    ],
)(_gather_body)


@jax.jit
def kernel(inputs, shared_weights):
    idx = inputs.reshape(B_TOTAL).astype(jnp.int32)
    return _gather(idx, shared_weights)
